# Initial kernel scaffold; baseline (speedup 1.0000x reference)
import jax
import jax.numpy as jnp
from jax.experimental import pallas as pl


def kernel(x, edge_index, W1, b1, gamma, beta, W2, b2):
    raise NotImplementedError("write your pallas kernel here")


# speedup vs baseline: 9.5151x; 9.5151x over previous
import functools

import jax
import jax.numpy as jnp
from jax import lax
from jax.experimental import pallas as pl
from jax.experimental.pallas import tpu as pltpu
from jax.experimental.pallas import tpu_sc as plsc

N = 10000
D = 128
E = 320000
NC = 2
NS = 16
L = 16
N_PAD = 10112
ROWS_PER_TILE = N_PAD // NS
CHUNK = 128
E_PAD = 327680
NUM_CHUNKS = E_PAD // CHUNK
CHUNKS_PER_TILE = NUM_CHUNKS // (NC * NS)
BLK = 400

_mesh = plsc.VectorSubcoreMesh(core_axis_name="c", subcore_axis_name="s")


def _fill(buf, val, nrows, width):
    vec = jnp.full((width,), val, jnp.float32)

    def body(i, carry):
        buf[i, :] = vec
        return carry

    lax.fori_loop(0, nrows, body, 0)


@functools.partial(
    pl.kernel,
    out_type=jax.ShapeDtypeStruct((NC, N_PAD, L), jnp.float32),
    mesh=_mesh,
    scratch_types=[
        pltpu.VMEM_SHARED((N_PAD, L), jnp.float32),
        pltpu.VMEM((CHUNKS_PER_TILE, CHUNK), jnp.int32),
        pltpu.VMEM((CHUNK, L), jnp.float32),
    ],
)
def _deg_partials(dst_hbm, out_hbm, acc_sh, idx_v, buf_v):
    c = lax.axis_index("c")
    s = lax.axis_index("s")
    wid = c * NS + s
    base = s * ROWS_PER_TILE
    _fill(buf_v, 0.0, CHUNK, L)
    for k in range(4):
        pltpu.sync_copy(buf_v, acc_sh.at[pl.ds(base + k * CHUNK, CHUNK)])
    pltpu.sync_copy(buf_v.at[pl.ds(0, ROWS_PER_TILE - 4 * CHUNK)],
                    acc_sh.at[pl.ds(base + 4 * CHUNK, ROWS_PER_TILE - 4 * CHUNK)])
    pltpu.sync_copy(dst_hbm.at[pl.ds(wid * CHUNKS_PER_TILE, CHUNKS_PER_TILE)], idx_v)
    plsc.subcore_barrier()
    _fill(buf_v, 1.0, CHUNK, L)

    def body(j, carry):
        pltpu.sync_copy(buf_v, acc_sh.at[idx_v.at[j]], add=True)
        return carry

    lax.fori_loop(0, CHUNKS_PER_TILE, body, 0)
    plsc.subcore_barrier()
    pltpu.sync_copy(acc_sh.at[pl.ds(base, ROWS_PER_TILE)],
                    out_hbm.at[c, pl.ds(base, ROWS_PER_TILE)])


@functools.partial(
    pl.kernel,
    out_type=jax.ShapeDtypeStruct((NC, N_PAD, D), jnp.float32),
    mesh=_mesh,
    scratch_types=[
        pltpu.VMEM_SHARED((N_PAD, D), jnp.float32),
        pltpu.VMEM((CHUNKS_PER_TILE, CHUNK), jnp.int32),
        pltpu.VMEM((CHUNKS_PER_TILE, CHUNK), jnp.int32),
        pltpu.VMEM((CHUNK, D), jnp.float32),
        pltpu.SemaphoreType.DMA,
    ],
)
def _agg_partials(y_hbm, src_hbm, dst_hbm, out_hbm, acc_sh, src_v, dst_v, rows_v, sem):
    c = lax.axis_index("c")
    s = lax.axis_index("s")
    wid = c * NS + s
    base = s * ROWS_PER_TILE
    pltpu.sync_copy(y_hbm.at[pl.ds(base, ROWS_PER_TILE)],
                    acc_sh.at[pl.ds(base, ROWS_PER_TILE)])
    eb = wid * CHUNKS_PER_TILE
    pltpu.sync_copy(src_hbm.at[pl.ds(eb, CHUNKS_PER_TILE)], src_v)
    pltpu.sync_copy(dst_hbm.at[pl.ds(eb, CHUNKS_PER_TILE)], dst_v)
    plsc.subcore_barrier()

    def body(j, carry):
        pltpu.async_copy(y_hbm.at[src_v.at[j]], rows_v, sem).wait()
        pltpu.sync_copy(rows_v, acc_sh.at[dst_v.at[j]], add=True)
        return carry

    lax.fori_loop(0, CHUNKS_PER_TILE, body, 0)
    plsc.subcore_barrier()
    pltpu.sync_copy(acc_sh.at[pl.ds(base, ROWS_PER_TILE)],
                    out_hbm.at[c, pl.ds(base, ROWS_PER_TILE)])


def _dinv_of(p0, p1):
    return lax.rsqrt(p0[:, :1] + p1[:, :1] + 1.0)


def _tc_y_body(x_ref, w_ref, p0_ref, p1_ref, y_ref):
    dinv = _dinv_of(p0_ref[...], p1_ref[...])
    y_ref[...] = jnp.dot(x_ref[...], w_ref[...],
                         preferred_element_type=jnp.float32) * dinv


def _tc_mid_body(p0_ref, p1_ref, a0_ref, a1_ref, y1_ref, b1_ref, g_ref, be_ref,
                 w2_ref, y2_ref):
    dinv = _dinv_of(p0_ref[...], p1_ref[...])
    h = dinv * (a0_ref[...] + a1_ref[...] - y1_ref[...]) + b1_ref[...]
    bn_scale = jnp.float32(1.0 / (1.0 + 1e-5) ** 0.5)
    h = h * (g_ref[...] * bn_scale) + be_ref[...]
    h = jnp.maximum(h, 0.0)
    y2_ref[...] = jnp.dot(h, w2_ref[...], preferred_element_type=jnp.float32) * dinv


def _tc_out_body(p0_ref, p1_ref, a0_ref, a1_ref, y2_ref, b2_ref, o_ref):
    dinv = _dinv_of(p0_ref[...], p1_ref[...])
    o = dinv * (a0_ref[...] + a1_ref[...] - y2_ref[...]) + b2_ref[...]
    m = jnp.max(o, axis=1, keepdims=True)
    ex = jnp.exp(o - m)
    o_ref[...] = (o - m) - jnp.log(jnp.sum(ex, axis=1, keepdims=True))


def _row_spec(w):
    return pl.BlockSpec((BLK, w), lambda i: (i, 0))


def _full_spec(shape):
    return pl.BlockSpec(shape, lambda i: (0, 0))


_tc_y = pl.pallas_call(
    _tc_y_body,
    grid=(N // BLK,),
    in_specs=[_row_spec(D), _full_spec((D, D)), _row_spec(L), _row_spec(L)],
    out_specs=_row_spec(D),
    out_shape=jax.ShapeDtypeStruct((N, D), jnp.float32),
)

_tc_mid = pl.pallas_call(
    _tc_mid_body,
    grid=(N // BLK,),
    in_specs=[_row_spec(L), _row_spec(L), _row_spec(D), _row_spec(D), _row_spec(D),
              _full_spec((1, D)), _full_spec((1, D)), _full_spec((1, D)),
              _full_spec((D, D))],
    out_specs=_row_spec(D),
    out_shape=jax.ShapeDtypeStruct((N, D), jnp.float32),
)

_tc_out = pl.pallas_call(
    _tc_out_body,
    grid=(N // BLK,),
    in_specs=[_row_spec(L), _row_spec(L), _row_spec(D), _row_spec(D), _row_spec(D),
              _full_spec((1, D))],
    out_specs=_row_spec(D),
    out_shape=jax.ShapeDtypeStruct((N, D), jnp.float32),
)


def kernel(x, edge_index, W1, b1, gamma, beta, W2, b2):
    src = edge_index[0].astype(jnp.int32)
    dst = edge_index[1].astype(jnp.int32)
    pad = E_PAD - E
    src_p = jnp.concatenate([src, jnp.zeros((pad,), jnp.int32)]).reshape(NUM_CHUNKS, CHUNK)
    dst_p = jnp.concatenate([dst, jnp.full((pad,), N, jnp.int32)]).reshape(NUM_CHUNKS, CHUNK)

    degp = _deg_partials(dst_p)
    p0 = degp[0, :N]
    p1 = degp[1, :N]

    y1 = _tc_y(x, W1, p0, p1)
    y1p = jnp.concatenate([y1, jnp.zeros((N_PAD - N, D), jnp.float32)])
    a = _agg_partials(y1p, src_p, dst_p)

    y2 = _tc_mid(p0, p1, a[0, :N], a[1, :N], y1,
                 b1.reshape(1, D), gamma.reshape(1, D), beta.reshape(1, D), W2)
    y2p = jnp.concatenate([y2, jnp.zeros((N_PAD - N, D), jnp.float32)])
    a2 = _agg_partials(y2p, src_p, dst_p)

    return _tc_out(p0, p1, a2[0, :N], a2[1, :N], y2, b2.reshape(1, D))


# speedup vs baseline: 10.9156x; 1.1472x over previous
import functools

import jax
import jax.numpy as jnp
from jax import lax
from jax.experimental import pallas as pl
from jax.experimental.pallas import tpu as pltpu
from jax.experimental.pallas import tpu_sc as plsc

N = 10000
D = 128
E = 320000
NC = 2
NS = 16
L = 16
N_PAD = 10112
ROWS_PER_TILE = N_PAD // NS
CHUNK = 128
E_PAD = 327680
NUM_CHUNKS = E_PAD // CHUNK
CHUNKS_PER_TILE = NUM_CHUNKS // (NC * NS)
BLK = 400

_mesh = plsc.VectorSubcoreMesh(core_axis_name="c", subcore_axis_name="s")


def _fill(buf, val, nrows, width):
    vec = jnp.full((width,), val, jnp.float32)

    def body(i, carry):
        buf[i, :] = vec
        return carry

    lax.fori_loop(0, nrows, body, 0)


@functools.partial(
    pl.kernel,
    out_type=jax.ShapeDtypeStruct((NC, N_PAD, L), jnp.float32),
    mesh=_mesh,
    scratch_types=[
        pltpu.VMEM_SHARED((N_PAD, L), jnp.float32),
        pltpu.VMEM((CHUNKS_PER_TILE, CHUNK), jnp.int32),
        pltpu.VMEM((CHUNK, L), jnp.float32),
    ],
)
def _deg_partials(dst_hbm, out_hbm, acc_sh, idx_v, buf_v):
    c = lax.axis_index("c")
    s = lax.axis_index("s")
    wid = c * NS + s
    base = s * ROWS_PER_TILE
    _fill(buf_v, 0.0, CHUNK, L)
    nfull = ROWS_PER_TILE // CHUNK
    rem = ROWS_PER_TILE % CHUNK
    for k in range(nfull):
        pltpu.sync_copy(buf_v, acc_sh.at[pl.ds(base + k * CHUNK, CHUNK)])
    if rem:
        pltpu.sync_copy(buf_v.at[pl.ds(0, rem)],
                        acc_sh.at[pl.ds(base + nfull * CHUNK, rem)])
    pltpu.sync_copy(dst_hbm.at[pl.ds(wid * CHUNKS_PER_TILE, CHUNKS_PER_TILE)], idx_v)
    plsc.subcore_barrier()
    _fill(buf_v, 1.0, CHUNK, L)

    def body(j, carry):
        pltpu.sync_copy(buf_v, acc_sh.at[idx_v.at[j]], add=True)
        return carry

    lax.fori_loop(0, CHUNKS_PER_TILE, body, 0)
    plsc.subcore_barrier()
    pltpu.sync_copy(acc_sh.at[pl.ds(base, ROWS_PER_TILE)],
                    out_hbm.at[c, pl.ds(base, ROWS_PER_TILE)])


NBUF = 2
GROUPS_PER_TILE = 2
CHUNKS_PER_GROUP = CHUNKS_PER_TILE // GROUPS_PER_TILE


@functools.partial(
    pl.kernel,
    out_type=jax.ShapeDtypeStruct((NC, N_PAD, D), jnp.float32),
    mesh=_mesh,
    scratch_types=[
        pltpu.VMEM_SHARED((N_PAD, D), jnp.float32),
        pltpu.VMEM((CHUNKS_PER_GROUP, CHUNK), jnp.int32),
        pltpu.VMEM((CHUNKS_PER_GROUP, CHUNK), jnp.int32),
        [pltpu.VMEM((CHUNK, D), jnp.float32)] * NBUF,
        [pltpu.SemaphoreType.DMA] * NBUF,
        [pltpu.SemaphoreType.DMA] * NBUF,
    ],
)
def _agg_partials(y_hbm, src_hbm, dst_hbm, out_hbm, acc_sh, src_v, dst_v,
                  bufs, gsem, ssem):
    c = lax.axis_index("c")
    s = lax.axis_index("s")
    wid = c * NS + s
    base = s * ROWS_PER_TILE
    pltpu.sync_copy(y_hbm.at[pl.ds(base, ROWS_PER_TILE)],
                    acc_sh.at[pl.ds(base, ROWS_PER_TILE)])
    plsc.subcore_barrier()

    for g in range(GROUPS_PER_TILE):
        eb = wid * CHUNKS_PER_TILE + g * CHUNKS_PER_GROUP
        pltpu.sync_copy(src_hbm.at[pl.ds(eb, CHUNKS_PER_GROUP)], src_v)
        pltpu.sync_copy(dst_hbm.at[pl.ds(eb, CHUNKS_PER_GROUP)], dst_v)

        for k in range(NBUF - 1):
            pltpu.async_copy(y_hbm.at[src_v.at[k]], bufs[k], gsem[k])

        def body(t, carry):
            for k in range(NBUF):
                j = NBUF * t + k
                bcur = k
                bpre = (k + NBUF - 1) % NBUF

                @pl.when(j >= 1)
                def _wait_prev_scatter():
                    jp = lax.max(j - 1, 0)
                    pltpu.make_async_copy(bufs[bpre], acc_sh.at[dst_v.at[jp]],
                                          ssem[bpre]).wait()

                @pl.when(j + NBUF - 1 < CHUNKS_PER_GROUP)
                def _prefetch():
                    pltpu.async_copy(y_hbm.at[src_v.at[j + NBUF - 1]],
                                     bufs[bpre], gsem[bpre])

                pltpu.make_async_copy(y_hbm.at[src_v.at[j]], bufs[bcur],
                                      gsem[bcur]).wait()
                pltpu.async_copy(bufs[bcur], acc_sh.at[dst_v.at[j]],
                                 ssem[bcur], add=True)
            return carry

        lax.fori_loop(0, CHUNKS_PER_GROUP // NBUF, body, 0)
        last = CHUNKS_PER_GROUP - 1
        pltpu.make_async_copy(bufs[last % NBUF], acc_sh.at[dst_v.at[last]],
                              ssem[last % NBUF]).wait()
    plsc.subcore_barrier()
    pltpu.sync_copy(acc_sh.at[pl.ds(base, ROWS_PER_TILE)],
                    out_hbm.at[c, pl.ds(base, ROWS_PER_TILE)])


def _dinv_of(p0, p1):
    return lax.rsqrt(p0[:, :1] + p1[:, :1] + 1.0)


def _tc_y_body(x_ref, w_ref, p0_ref, p1_ref, y_ref):
    dinv = _dinv_of(p0_ref[...], p1_ref[...])
    y_ref[...] = jnp.dot(x_ref[...], w_ref[...],
                         preferred_element_type=jnp.float32) * dinv


def _tc_mid_body(p0_ref, p1_ref, a0_ref, a1_ref, y1_ref, b1_ref, g_ref, be_ref,
                 w2_ref, y2_ref):
    dinv = _dinv_of(p0_ref[...], p1_ref[...])
    h = dinv * (a0_ref[...] + a1_ref[...] - y1_ref[...]) + b1_ref[...]
    bn_scale = jnp.float32(1.0 / (1.0 + 1e-5) ** 0.5)
    h = h * (g_ref[...] * bn_scale) + be_ref[...]
    h = jnp.maximum(h, 0.0)
    y2_ref[...] = jnp.dot(h, w2_ref[...], preferred_element_type=jnp.float32) * dinv


def _tc_out_body(p0_ref, p1_ref, a0_ref, a1_ref, y2_ref, b2_ref, o_ref):
    dinv = _dinv_of(p0_ref[...], p1_ref[...])
    o = dinv * (a0_ref[...] + a1_ref[...] - y2_ref[...]) + b2_ref[...]
    m = jnp.max(o, axis=1, keepdims=True)
    ex = jnp.exp(o - m)
    o_ref[...] = (o - m) - jnp.log(jnp.sum(ex, axis=1, keepdims=True))


def _row_spec(w):
    return pl.BlockSpec((BLK, w), lambda i: (i, 0))


def _full_spec(shape):
    return pl.BlockSpec(shape, lambda i: (0, 0))


_tc_y = pl.pallas_call(
    _tc_y_body,
    grid=(N // BLK,),
    in_specs=[_row_spec(D), _full_spec((D, D)), _row_spec(L), _row_spec(L)],
    out_specs=_row_spec(D),
    out_shape=jax.ShapeDtypeStruct((N, D), jnp.float32),
)

_tc_mid = pl.pallas_call(
    _tc_mid_body,
    grid=(N // BLK,),
    in_specs=[_row_spec(L), _row_spec(L), _row_spec(D), _row_spec(D), _row_spec(D),
              _full_spec((1, D)), _full_spec((1, D)), _full_spec((1, D)),
              _full_spec((D, D))],
    out_specs=_row_spec(D),
    out_shape=jax.ShapeDtypeStruct((N, D), jnp.float32),
)

_tc_out = pl.pallas_call(
    _tc_out_body,
    grid=(N // BLK,),
    in_specs=[_row_spec(L), _row_spec(L), _row_spec(D), _row_spec(D), _row_spec(D),
              _full_spec((1, D))],
    out_specs=_row_spec(D),
    out_shape=jax.ShapeDtypeStruct((N, D), jnp.float32),
)


def kernel(x, edge_index, W1, b1, gamma, beta, W2, b2):
    src = edge_index[0].astype(jnp.int32)
    dst = edge_index[1].astype(jnp.int32)
    pad = E_PAD - E
    src_p = jnp.concatenate([src, jnp.zeros((pad,), jnp.int32)]).reshape(NUM_CHUNKS, CHUNK)
    dst_p = jnp.concatenate([dst, jnp.full((pad,), N, jnp.int32)]).reshape(NUM_CHUNKS, CHUNK)

    degp = _deg_partials(dst_p)
    p0 = degp[0, :N]
    p1 = degp[1, :N]

    y1 = _tc_y(x, W1, p0, p1)
    y1p = jnp.concatenate([y1, jnp.zeros((N_PAD - N, D), jnp.float32)])
    a = _agg_partials(y1p, src_p, dst_p)

    y2 = _tc_mid(p0, p1, a[0, :N], a[1, :N], y1,
                 b1.reshape(1, D), gamma.reshape(1, D), beta.reshape(1, D), W2)
    y2p = jnp.concatenate([y2, jnp.zeros((N_PAD - N, D), jnp.float32)])
    a2 = _agg_partials(y2p, src_p, dst_p)

    return _tc_out(p0, p1, a2[0, :N], a2[1, :N], y2, b2.reshape(1, D))


# speedup vs baseline: 11.0929x; 1.0162x over previous
import functools

import jax
import jax.numpy as jnp
from jax import lax
from jax.experimental import pallas as pl
from jax.experimental.pallas import tpu as pltpu
from jax.experimental.pallas import tpu_sc as plsc

N = 10000
D = 128
E = 320000
NC = 2
NS = 16
L = 16
N_PAD = 10112
ROWS_PER_TILE = N_PAD // NS
CHUNK = 128
E_PAD = 327680
NUM_CHUNKS = E_PAD // CHUNK
CHUNKS_PER_TILE = NUM_CHUNKS // (NC * NS)
BLK = 400

_mesh = plsc.VectorSubcoreMesh(core_axis_name="c", subcore_axis_name="s")


def _fill(buf, val, nrows, width):
    vec = jnp.full((width,), val, jnp.float32)

    def body(i, carry):
        buf[i, :] = vec
        return carry

    lax.fori_loop(0, nrows, body, 0)


@functools.partial(
    pl.kernel,
    out_type=jax.ShapeDtypeStruct((NC, N_PAD, L), jnp.float32),
    mesh=_mesh,
    scratch_types=[
        pltpu.VMEM_SHARED((N_PAD, L), jnp.float32),
        pltpu.VMEM((CHUNKS_PER_TILE, CHUNK), jnp.int32),
        pltpu.VMEM((CHUNK, L), jnp.float32),
    ],
)
def _deg_partials(dst_hbm, out_hbm, acc_sh, idx_v, buf_v):
    c = lax.axis_index("c")
    s = lax.axis_index("s")
    wid = c * NS + s
    base = s * ROWS_PER_TILE
    _fill(buf_v, 0.0, CHUNK, L)
    nfull = ROWS_PER_TILE // CHUNK
    rem = ROWS_PER_TILE % CHUNK
    for k in range(nfull):
        pltpu.sync_copy(buf_v, acc_sh.at[pl.ds(base + k * CHUNK, CHUNK)])
    if rem:
        pltpu.sync_copy(buf_v.at[pl.ds(0, rem)],
                        acc_sh.at[pl.ds(base + nfull * CHUNK, rem)])
    pltpu.sync_copy(dst_hbm.at[pl.ds(wid * CHUNKS_PER_TILE, CHUNKS_PER_TILE)], idx_v)
    plsc.subcore_barrier()
    _fill(buf_v, 1.0, CHUNK, L)

    def body(j, carry):
        pltpu.sync_copy(buf_v, acc_sh.at[idx_v.at[j]], add=True)
        return carry

    lax.fori_loop(0, CHUNKS_PER_TILE, body, 0)
    plsc.subcore_barrier()
    pltpu.sync_copy(acc_sh.at[pl.ds(base, ROWS_PER_TILE)],
                    out_hbm.at[c, pl.ds(base, ROWS_PER_TILE)])


NBUF = 2
GROUP = 40
CPT0 = 120
CPT1 = 40


@functools.partial(
    pl.kernel,
    out_type=jax.ShapeDtypeStruct((NC, N_PAD, D), jnp.float32),
    mesh=_mesh,
    scratch_types=[
        pltpu.VMEM_SHARED((N_PAD, D), jnp.float32),
        pltpu.VMEM((GROUP, CHUNK), jnp.int32),
        pltpu.VMEM((GROUP, CHUNK), jnp.int32),
        [pltpu.VMEM((CHUNK, D), jnp.float32)] * NBUF,
        [pltpu.SemaphoreType.DMA] * NBUF,
        [pltpu.SemaphoreType.DMA] * NBUF,
    ],
)
def _agg_partials(y_hbm, src_hbm, dst_hbm, out_hbm, acc_sh, src_v, dst_v,
                  bufs, gsem, ssem):
    c = lax.axis_index("c")
    s = lax.axis_index("s")
    wid = c * NS + s
    base = s * ROWS_PER_TILE
    pltpu.sync_copy(y_hbm.at[pl.ds(base, ROWS_PER_TILE)],
                    acc_sh.at[pl.ds(base, ROWS_PER_TILE)])
    plsc.subcore_barrier()

    cpt = jnp.where(c == 0, CPT0, CPT1)
    tile_base = c * (NS * CPT0) + s * cpt

    def group_body(g, carry):
        eb = pl.multiple_of(tile_base + g * GROUP, 8)
        pltpu.sync_copy(src_hbm.at[pl.ds(eb, GROUP)], src_v)
        pltpu.sync_copy(dst_hbm.at[pl.ds(eb, GROUP)], dst_v)

        for k in range(NBUF - 1):
            pltpu.async_copy(y_hbm.at[src_v.at[k]], bufs[k], gsem[k])

        def body(t, carry2):
            for k in range(NBUF):
                j = NBUF * t + k
                bcur = k
                bpre = (k + NBUF - 1) % NBUF

                @pl.when(j >= 1)
                def _wait_prev_scatter():
                    jp = lax.max(j - 1, 0)
                    pltpu.make_async_copy(bufs[bpre], acc_sh.at[dst_v.at[jp]],
                                          ssem[bpre]).wait()

                @pl.when(j + NBUF - 1 < GROUP)
                def _prefetch():
                    pltpu.async_copy(y_hbm.at[src_v.at[j + NBUF - 1]],
                                     bufs[bpre], gsem[bpre])

                pltpu.make_async_copy(y_hbm.at[src_v.at[j]], bufs[bcur],
                                      gsem[bcur]).wait()
                pltpu.async_copy(bufs[bcur], acc_sh.at[dst_v.at[j]],
                                 ssem[bcur], add=True)
            return carry2

        lax.fori_loop(0, GROUP // NBUF, body, 0)
        last = GROUP - 1
        pltpu.make_async_copy(bufs[last % NBUF], acc_sh.at[dst_v.at[last]],
                              ssem[last % NBUF]).wait()
        return carry

    lax.fori_loop(0, cpt // GROUP, group_body, 0)
    plsc.subcore_barrier()
    pltpu.sync_copy(acc_sh.at[pl.ds(base, ROWS_PER_TILE)],
                    out_hbm.at[c, pl.ds(base, ROWS_PER_TILE)])


def _dinv_of(p0, p1):
    return lax.rsqrt(p0[:, :1] + p1[:, :1] + 1.0)


def _tc_y_body(x_ref, w_ref, p0_ref, p1_ref, y_ref):
    dinv = _dinv_of(p0_ref[...], p1_ref[...])
    y_ref[...] = jnp.dot(x_ref[...], w_ref[...],
                         preferred_element_type=jnp.float32) * dinv


def _tc_mid_body(p0_ref, p1_ref, a0_ref, a1_ref, y1_ref, b1_ref, g_ref, be_ref,
                 w2_ref, y2_ref):
    dinv = _dinv_of(p0_ref[...], p1_ref[...])
    h = dinv * (a0_ref[...] + a1_ref[...] - y1_ref[...]) + b1_ref[...]
    bn_scale = jnp.float32(1.0 / (1.0 + 1e-5) ** 0.5)
    h = h * (g_ref[...] * bn_scale) + be_ref[...]
    h = jnp.maximum(h, 0.0)
    y2_ref[...] = jnp.dot(h, w2_ref[...], preferred_element_type=jnp.float32) * dinv


def _tc_out_body(p0_ref, p1_ref, a0_ref, a1_ref, y2_ref, b2_ref, o_ref):
    dinv = _dinv_of(p0_ref[...], p1_ref[...])
    o = dinv * (a0_ref[...] + a1_ref[...] - y2_ref[...]) + b2_ref[...]
    m = jnp.max(o, axis=1, keepdims=True)
    ex = jnp.exp(o - m)
    o_ref[...] = (o - m) - jnp.log(jnp.sum(ex, axis=1, keepdims=True))


def _row_spec(w):
    return pl.BlockSpec((BLK, w), lambda i: (i, 0))


def _full_spec(shape):
    return pl.BlockSpec(shape, lambda i: (0, 0))


_tc_y = pl.pallas_call(
    _tc_y_body,
    grid=(N // BLK,),
    in_specs=[_row_spec(D), _full_spec((D, D)), _row_spec(L), _row_spec(L)],
    out_specs=_row_spec(D),
    out_shape=jax.ShapeDtypeStruct((N, D), jnp.float32),
)

_tc_mid = pl.pallas_call(
    _tc_mid_body,
    grid=(N // BLK,),
    in_specs=[_row_spec(L), _row_spec(L), _row_spec(D), _row_spec(D), _row_spec(D),
              _full_spec((1, D)), _full_spec((1, D)), _full_spec((1, D)),
              _full_spec((D, D))],
    out_specs=_row_spec(D),
    out_shape=jax.ShapeDtypeStruct((N, D), jnp.float32),
)

_tc_out = pl.pallas_call(
    _tc_out_body,
    grid=(N // BLK,),
    in_specs=[_row_spec(L), _row_spec(L), _row_spec(D), _row_spec(D), _row_spec(D),
              _full_spec((1, D))],
    out_specs=_row_spec(D),
    out_shape=jax.ShapeDtypeStruct((N, D), jnp.float32),
)


def kernel(x, edge_index, W1, b1, gamma, beta, W2, b2):
    src = edge_index[0].astype(jnp.int32)
    dst = edge_index[1].astype(jnp.int32)
    pad = E_PAD - E
    src_p = jnp.concatenate([src, jnp.zeros((pad,), jnp.int32)]).reshape(NUM_CHUNKS, CHUNK)
    dst_p = jnp.concatenate([dst, jnp.full((pad,), N, jnp.int32)]).reshape(NUM_CHUNKS, CHUNK)

    degp = _deg_partials(dst_p)
    p0 = degp[0, :N]
    p1 = degp[1, :N]

    y1 = _tc_y(x, W1, p0, p1)
    y1p = jnp.concatenate([y1, jnp.zeros((N_PAD - N, D), jnp.float32)])
    a = _agg_partials(y1p, src_p, dst_p)

    y2 = _tc_mid(p0, p1, a[0, :N], a[1, :N], y1,
                 b1.reshape(1, D), gamma.reshape(1, D), beta.reshape(1, D), W2)
    y2p = jnp.concatenate([y2, jnp.zeros((N_PAD - N, D), jnp.float32)])
    a2 = _agg_partials(y2p, src_p, dst_p)

    return _tc_out(p0, p1, a2[0, :N], a2[1, :N], y2, b2.reshape(1, D))
